# baseline (device time: 121525 ns/iter reference)
import jax
import jax.numpy as jnp
from jax import lax
from jax.experimental import pallas as pl
from jax.experimental.pallas import tpu as pltpu

N_DEV = 4


def kernel(x, w_mat, scale_x, scale_w):
    m_per, k = x.shape
    _, n = w_mat.shape
    half = m_per // 2
    n_slots = 4

    def body(x_ref, w_ref, sx_ref, sw_ref, out_hbm,
             fl_ref, fr_ref, opp_ref, y_ref,
             send_sems, recv_sems, copy_sems):
        my = lax.axis_index("i")
        left = (my - 1) % N_DEV
        right = (my + 1) % N_DEV
        opp = (my + 2) % N_DEV

        barrier_sem = pltpu.get_barrier_semaphore()
        for nbr in [left, right]:
            pl.semaphore_signal(
                barrier_sem, inc=1,
                device_id=(nbr,), device_id_type=pl.DeviceIdType.MESH,
            )
        pl.semaphore_wait(barrier_sem, 2)

        scale = sx_ref[0] * sw_ref[0]
        top = pl.ds(0, half)
        bot = pl.ds(half, half)

        copies = []

        def compute(chunk, row_block):
            idx = len(copies)
            slot = idx % n_slots
            if idx >= n_slots:
                copies[idx - n_slots].wait()
            acc = jnp.dot(chunk, w_ref[...], preferred_element_type=jnp.int32)
            y = acc.astype(jnp.float32) * scale
            y_ref[slot] = y * jax.nn.sigmoid(y)
            cp = pltpu.make_async_copy(
                y_ref.at[slot],
                out_hbm.at[pl.ds(row_block * half, half), :],
                copy_sems.at[slot],
            )
            cp.start()
            copies.append(cp)

        def rdma(src, dst, i, dev):
            return pltpu.make_async_remote_copy(
                src_ref=src, dst_ref=dst,
                send_sem=send_sems.at[i], recv_sem=recv_sems.at[i],
                device_id=(dev,), device_id_type=pl.DeviceIdType.MESH,
            )

        send_r_top = rdma(x_ref.at[top, :], fl_ref.at[top, :], 0, right)
        send_r_bot = rdma(x_ref.at[bot, :], fl_ref.at[bot, :], 1, right)
        send_l_bot = rdma(x_ref.at[bot, :], fr_ref.at[bot, :], 2, left)
        send_l_top = rdma(x_ref.at[top, :], fr_ref.at[top, :], 3, left)
        fwd_r = rdma(fl_ref.at[top, :], opp_ref.at[top, :], 4, right)
        fwd_l = rdma(fr_ref.at[bot, :], opp_ref.at[bot, :], 5, left)

        for s in (send_r_top, send_r_bot, send_l_bot, send_l_top):
            s.start()

        compute(x_ref[top, :], 2 * my)
        compute(x_ref[bot, :], 2 * my + 1)

        send_r_top.wait_recv()
        fwd_r.start()
        send_l_bot.wait_recv()
        fwd_l.start()
        compute(fl_ref[top, :], 2 * left)
        compute(fr_ref[bot, :], 2 * right + 1)

        send_r_bot.wait_recv()
        send_l_top.wait_recv()
        compute(fl_ref[bot, :], 2 * left + 1)
        compute(fr_ref[top, :], 2 * right)

        fwd_r.wait_recv()
        fwd_l.wait_recv()
        compute(opp_ref[top, :], 2 * opp)
        compute(opp_ref[bot, :], 2 * opp + 1)

        for s in (send_r_top, send_r_bot, send_l_bot, send_l_top,
                  fwd_r, fwd_l):
            s.wait_send()
        for cp in copies[-n_slots:]:
            cp.wait()

    out_shape = jax.ShapeDtypeStruct((N_DEV * m_per, n), jnp.float32)
    return pl.pallas_call(
        body,
        out_shape=out_shape,
        in_specs=[
            pl.BlockSpec(memory_space=pltpu.VMEM),
            pl.BlockSpec(memory_space=pltpu.VMEM),
            pl.BlockSpec(memory_space=pltpu.SMEM),
            pl.BlockSpec(memory_space=pltpu.SMEM),
        ],
        out_specs=pl.BlockSpec(memory_space=pl.ANY),
        scratch_shapes=[
            pltpu.VMEM((m_per, k), x.dtype),
            pltpu.VMEM((m_per, k), x.dtype),
            pltpu.VMEM((m_per, k), x.dtype),
            pltpu.VMEM((n_slots, half, n), jnp.float32),
            pltpu.SemaphoreType.DMA((6,)),
            pltpu.SemaphoreType.DMA((6,)),
            pltpu.SemaphoreType.DMA((n_slots,)),
        ],
        compiler_params=pltpu.CompilerParams(
            collective_id=0,
            vmem_limit_bytes=60 * 1024 * 1024,
        ),
    )(x, w_mat, scale_x, scale_w)
